# baseline (speedup 1.0000x reference)
import jax
import jax.numpy as jnp
from jax.experimental import pallas as pl
from jax.experimental.pallas import tpu as pltpu

_TOPK = 8
_TEMP = 0.1
_EPS = 1e-07


def _fused_body(z_ref, p_ref, sim_ref, out_ref):
    bb, l, d = z_ref.shape
    k = p_ref.shape[0]
    with jax.named_scope("norm"):
        z3 = z_ref[...]
        p = p_ref[...]
        zn3 = z3 / (jnp.sqrt(jnp.sum(z3 * z3, axis=-1, keepdims=True)) + _EPS)
        pn = p / (jnp.sqrt(jnp.sum(p * p, axis=-1, keepdims=True)) + _EPS)
        zn = zn3.astype(jnp.bfloat16).reshape(bb * l, d)
        pn = pn.astype(jnp.bfloat16)
    with jax.named_scope("simmm"):
        sim = jax.lax.dot_general(
            zn, pn, (((1,), (1,)), ((), ())),
            preferred_element_type=jnp.float32,
        ) * (1.0 / _TEMP)
        sim_ref[...] = sim.reshape(bb, l, k)

    with jax.named_scope("topk"):
        m1 = jnp.max(sim, axis=-1, keepdims=True)
        cur = m1
        for _ in range(_TOPK - 1):
            cur = jnp.max(jnp.where(sim < cur, sim, -jnp.inf),
                          axis=-1, keepdims=True)
        t8 = cur

    with jax.named_scope("wout"):
        w = jnp.where(sim >= t8, jnp.exp(sim - m1), 0.0)
        denom = jnp.sum(w, axis=-1, keepdims=True)
        comb = jax.lax.dot_general(
            w.astype(jnp.bfloat16), p.astype(jnp.bfloat16),
            (((1,), (0,)), ((), ())),
            preferred_element_type=jnp.float32,
        )
        out_ref[...] = (comb / denom).reshape(bb, l, d)


def _pick_block(total: int, target: int) -> int:
    best = 1
    for t in range(1, target + 1):
        if total % t == 0:
            best = t
    return best


def kernel(z, proto_embs):
    b, l, d = z.shape
    k = proto_embs.shape[0]
    bb = _pick_block(b, 128)
    grid = (b // bb,)

    sim, out = pl.pallas_call(
        _fused_body,
        grid=grid,
        in_specs=[
            pl.BlockSpec((bb, l, d), lambda i: (i, 0, 0)),
            pl.BlockSpec((k, d), lambda i: (0, 0)),
        ],
        out_specs=[
            pl.BlockSpec((bb, l, k), lambda i: (i, 0, 0)),
            pl.BlockSpec((bb, l, d), lambda i: (i, 0, 0)),
        ],
        out_shape=[
            jax.ShapeDtypeStruct((b, l, k), jnp.float32),
            jax.ShapeDtypeStruct((b, l, d), jnp.float32),
        ],
        compiler_params=pltpu.CompilerParams(
            dimension_semantics=("parallel",),
            vmem_limit_bytes=120 * 1024 * 1024,
        ),
    )(z, proto_embs)
    return out, sim


# speedup vs baseline: 1.0126x; 1.0126x over previous
import jax
import jax.numpy as jnp
from jax.experimental import pallas as pl
from jax.experimental.pallas import tpu as pltpu

_TOPK = 8
_TEMP = 0.1
_EPS = 1e-07


def _fused_body(z_ref, p_ref, sim_ref, out_ref):
    bb, l, d = z_ref.shape
    k = p_ref.shape[0]
    with jax.named_scope("norm"):
        z3 = z_ref[...]
        p = p_ref[...]
        zn3 = z3 / (jnp.sqrt(jnp.sum(z3 * z3, axis=-1, keepdims=True)) + _EPS)
        pn = p / (jnp.sqrt(jnp.sum(p * p, axis=-1, keepdims=True)) + _EPS)
        zn = zn3.astype(jnp.bfloat16).reshape(bb * l, d)
        pn = pn.astype(jnp.bfloat16)
    with jax.named_scope("simmm"):
        sim = jax.lax.dot_general(
            zn, pn, (((1,), (1,)), ((), ())),
            preferred_element_type=jnp.float32,
        ) * (1.0 / _TEMP)
        sim_ref[...] = sim.reshape(bb, l, k)

    with jax.named_scope("topk"):
        m1 = jnp.max(sim, axis=-1, keepdims=True)
        cur = m1
        for _ in range(_TOPK - 1):
            cur = jnp.max(jnp.where(sim < cur, sim, -jnp.inf),
                          axis=-1, keepdims=True)
        t8 = cur

    with jax.named_scope("wout"):
        w = jnp.where(sim >= t8, jnp.exp(sim - m1), 0.0)
        denom = jnp.sum(w, axis=-1, keepdims=True)
        comb = jax.lax.dot_general(
            w, p, (((1,), (0,)), ((), ())),
            preferred_element_type=jnp.float32,
        )
        out_ref[...] = (comb / denom).reshape(bb, l, d)


def _pick_block(total: int, target: int) -> int:
    best = 1
    for t in range(1, target + 1):
        if total % t == 0:
            best = t
    return best


def kernel(z, proto_embs):
    b, l, d = z.shape
    k = proto_embs.shape[0]
    bb = _pick_block(b, 128)
    grid = (b // bb,)

    sim, out = pl.pallas_call(
        _fused_body,
        grid=grid,
        in_specs=[
            pl.BlockSpec((bb, l, d), lambda i: (i, 0, 0)),
            pl.BlockSpec((k, d), lambda i: (0, 0)),
        ],
        out_specs=[
            pl.BlockSpec((bb, l, k), lambda i: (i, 0, 0)),
            pl.BlockSpec((bb, l, d), lambda i: (i, 0, 0)),
        ],
        out_shape=[
            jax.ShapeDtypeStruct((b, l, k), jnp.float32),
            jax.ShapeDtypeStruct((b, l, d), jnp.float32),
        ],
        compiler_params=pltpu.CompilerParams(
            dimension_semantics=("parallel",),
            vmem_limit_bytes=120 * 1024 * 1024,
        ),
    )(z, proto_embs)
    return out, sim
